# baseline (speedup 1.0000x reference)
import functools

import jax
import jax.numpy as jnp
import numpy as np
from jax import lax
from jax.experimental import pallas as pl
from jax.experimental.pallas import tpu as pltpu
from jax.experimental.pallas import tpu_sc as plsc

_B = 16384
_BIT = 128
_ROWS = 2048

_SC_ROWS = 8192
_TC_ROWS = _B - _SC_ROWS
_NC, _NS = 2, 16
_NW = _NC * _NS
_RPW = _SC_ROWS // _NW

_K0 = np.uint32(0)
_K1 = np.uint32(42)
_KS2 = np.uint32(int(_K0) ^ int(_K1) ^ 0x1BD11BDA)
_ROT_A = (13, 15, 26, 6)
_ROT_B = (17, 29, 16, 24)


def _rotl(v, r):
    return (v << jnp.uint32(r)) | (v >> jnp.uint32(32 - r))


def _threefry_mix(x1):
    ks = (_K0, _K1, _KS2)
    rots = (_ROT_A, _ROT_B)
    x0 = x1
    x1 = _rotl(x1, rots[0][0]) ^ x0
    for r in rots[0][1:]:
        x0 = x0 + x1
        x1 = _rotl(x1, r)
        x1 = x1 ^ x0
    x0 = x0 + ks[1]
    x1 = x1 + ks[2] + jnp.uint32(1)
    for i in range(1, 5):
        for r in rots[i % 2]:
            x0 = x0 + x1
            x1 = _rotl(x1, r)
            x1 = x1 ^ x0
        x0 = x0 + ks[(i + 1) % 3]
        x1 = x1 + ks[(i + 2) % 3] + jnp.uint32(i + 1)
    return x0 ^ x1


def _tc_kernel(t_ref, o_ref, iota_ref):
    i = pl.program_id(0)

    @pl.when(i == 0)
    def _init():
        row = lax.broadcasted_iota(jnp.uint32, (_ROWS, _BIT), 0)
        col = lax.broadcasted_iota(jnp.uint32, (_ROWS, _BIT), 1)
        iota_ref[...] = row * jnp.uint32(_BIT) + col + jnp.uint32(int(_K1))

    base = jnp.uint32((_SC_ROWS + i * _ROWS) * _BIT)
    x1 = base + iota_ref[...]
    bits = _threefry_mix(x1)
    m = bits >> jnp.uint32(9)
    t = t_ref[...]
    o_ref[...] = jnp.where(m < t, 1.0, 0.0).astype(jnp.float32)


def _tc_bits(t2):
    return pl.pallas_call(
        _tc_kernel,
        grid=(_TC_ROWS // _ROWS,),
        in_specs=[pl.BlockSpec((_ROWS, 1), lambda i: (i, 0))],
        out_specs=pl.BlockSpec((_ROWS, _BIT), lambda i: (i, 0)),
        out_shape=jax.ShapeDtypeStruct((_TC_ROWS, _BIT), jnp.float32),
        scratch_shapes=[pltpu.VMEM((_ROWS, _BIT), jnp.uint32)],
        compiler_params=pltpu.CompilerParams(
            dimension_semantics=("arbitrary",),
        ),
    )(t2)


@functools.partial(
    pl.kernel,
    out_type=jax.ShapeDtypeStruct((_SC_ROWS * _BIT,), jnp.float32),
    mesh=plsc.VectorSubcoreMesh(core_axis_name="c", subcore_axis_name="s"),
    scratch_types=[
        pltpu.VMEM((_RPW * 16,), jnp.uint32),
        pltpu.VMEM((_RPW * _BIT,), jnp.float32),
    ],
)
def _sc_bits(texp_hbm, out_hbm, t_v, o_v):
    wid = lax.axis_index("c") * _NS + lax.axis_index("s")
    base_row = wid * _RPW
    pltpu.sync_copy(texp_hbm.at[pl.ds(base_row * 16, _RPW * 16)], t_v)
    lane = lax.iota(jnp.uint32, 16)

    def row_body(r, carry):
        t_vec = t_v[pl.ds(r * 16, 16)]
        gbase = (jnp.int32(base_row) + r) * _BIT
        lbase = r * _BIT
        for j in range(_BIT // 16):
            x1 = lane + jnp.uint32(gbase + (16 * j + int(_K1)))
            m = _threefry_mix(x1) >> jnp.uint32(9)
            val = jnp.where(m < t_vec, 1.0, 0.0).astype(jnp.float32)
            o_v[pl.ds(lbase + 16 * j, 16)] = val
        return carry

    lax.fori_loop(0, _RPW, row_body, 0, unroll=False)
    pltpu.sync_copy(o_v, out_hbm.at[pl.ds(base_row * _BIT, _RPW * _BIT)])


def kernel(x):
    t = jnp.ceil(x * jnp.float32(8388608.0)).astype(jnp.uint32)
    t_exp = jnp.repeat(t[:_SC_ROWS], 16)
    sc_out = _sc_bits(t_exp).reshape(_SC_ROWS, _BIT)
    tc_out = _tc_bits(t[_SC_ROWS:].reshape(_TC_ROWS, 1))
    return jnp.concatenate([sc_out, tc_out], axis=0)


# speedup vs baseline: 1.8878x; 1.8878x over previous
import jax
import jax.numpy as jnp
import numpy as np
from jax import lax
from jax.experimental import pallas as pl
from jax.experimental.pallas import tpu as pltpu

_B = 16384
_BIT = 128
_ROWS = 2048

_K0 = np.uint32(0)
_K1 = np.uint32(42)
_KS2 = np.uint32(int(_K0) ^ int(_K1) ^ 0x1BD11BDA)
_ROT_A = (13, 15, 26, 6)
_ROT_B = (17, 29, 16, 24)


def _rotl(v, r):
    return (v << jnp.uint32(r)) | (v >> jnp.uint32(32 - r))


def _threefry_mix(x1):
    ks = (_K0, _K1, _KS2)
    rots = (_ROT_A, _ROT_B)
    x0 = x1
    x1 = _rotl(x1, rots[0][0]) ^ x0
    for r in rots[0][1:]:
        x0 = x0 + x1
        x1 = _rotl(x1, r)
        x1 = x1 ^ x0
    x0 = x0 + ks[1]
    x1 = x1 + ks[2] + jnp.uint32(1)
    for i in range(1, 5):
        for r in rots[i % 2]:
            x0 = x0 + x1
            x1 = _rotl(x1, r)
            x1 = x1 ^ x0
        x0 = x0 + ks[(i + 1) % 3]
        x1 = x1 + ks[(i + 2) % 3] + jnp.uint32(i + 1)
    return x0 ^ x1


def _bits_kernel(t_ref, o_ref, iota_ref):
    i = pl.program_id(0)

    @pl.when(i == 0)
    def _init():
        row = lax.broadcasted_iota(jnp.uint32, (_ROWS, _BIT), 0)
        col = lax.broadcasted_iota(jnp.uint32, (_ROWS, _BIT), 1)
        iota_ref[...] = row * jnp.uint32(_BIT) + col + jnp.uint32(int(_K1))

    base = jnp.uint32(i * (_ROWS * _BIT))
    x1 = base + iota_ref[...]
    bits = _threefry_mix(x1)
    m = bits >> jnp.uint32(9)
    t = t_ref[...]
    o_ref[...] = jnp.where(m < t, 1.0, 0.0).astype(jnp.float32)


def kernel(x):
    t = jnp.ceil(x * jnp.float32(8388608.0)).astype(jnp.uint32).reshape(_B, 1)
    out = pl.pallas_call(
        _bits_kernel,
        grid=(_B // _ROWS,),
        in_specs=[pl.BlockSpec((_ROWS, 1), lambda i: (i, 0))],
        out_specs=pl.BlockSpec((_ROWS, _BIT), lambda i: (i, 0)),
        out_shape=jax.ShapeDtypeStruct((_B, _BIT), jnp.float32),
        scratch_shapes=[pltpu.VMEM((_ROWS, _BIT), jnp.uint32)],
        compiler_params=pltpu.CompilerParams(
            dimension_semantics=("arbitrary",),
        ),
    )(t)
    return out
